# baseline (speedup 1.0000x reference)
import jax
import jax.numpy as jnp
from jax.experimental import pallas as pl

P = 4096
B = 6
F_IN = 960
BP1 = 2048
BP2 = 512


def _stage1_body(emb_ref, refp_ref, w_emb_ref, w_refp_ref, b12_ref, out_ref):
    e = emb_ref[0, :, :]
    s = jnp.dot(e, w_emb_ref[:, :], preferred_element_type=jnp.float32)
    s = s + jnp.dot(refp_ref[:, :], w_refp_ref[:, :],
                    preferred_element_type=jnp.float32)
    out_ref[0, :, :] = s + b12_ref[0:1, :]


def _stage2_body(adj_ref, s36_ref, sself_ref, pm_ref, rgb_ref):
    res = jnp.dot(adj_ref[:, :], s36_ref[:, :],
                  preferred_element_type=jnp.float32)
    res = res + sself_ref[:, :]
    pm_ref[:, :] = jnp.tanh(res[:, 0:18])
    rgb_ref[:, :] = jax.nn.sigmoid(res[:, 18:36])


def kernel(embeddings, ref, adj, W_d, Wl_d, b_d, W_r, Wl_r, b_r):
    f32 = jnp.float32
    W12 = jnp.concatenate([W_d, W_r, Wl_d, Wl_r], axis=1).astype(f32)
    w_emb = W12[:F_IN, :]
    w_refp = jnp.pad(W12[F_IN:, :], ((0, 5), (0, 0)))
    refp = jnp.pad(ref[0].astype(f32), ((0, 0), (0, 5)))
    b12 = jnp.concatenate([jnp.zeros((6,), f32), b_d.astype(f32),
                           b_r.astype(f32)])
    b12 = jnp.tile(b12[None, :], (8, 1))

    nb1 = P // BP1
    s_all = pl.pallas_call(
        _stage1_body,
        grid=(B, nb1),
        in_specs=[
            pl.BlockSpec((1, BP1, F_IN), lambda b, i: (b, i, 0)),
            pl.BlockSpec((BP1, 8), lambda b, i: (i, 0)),
            pl.BlockSpec((F_IN, 12), lambda b, i: (0, 0)),
            pl.BlockSpec((8, 12), lambda b, i: (0, 0)),
            pl.BlockSpec((8, 12), lambda b, i: (0, 0)),
        ],
        out_specs=pl.BlockSpec((1, BP1, 12), lambda b, i: (b, i, 0)),
        out_shape=jax.ShapeDtypeStruct((B, P, 12), f32),
    )(embeddings.astype(f32), refp, w_emb, w_refp, b12)

    sd = s_all[:, :, 0:3].transpose(1, 0, 2).reshape(P, 18)
    sr = s_all[:, :, 3:6].transpose(1, 0, 2).reshape(P, 18)
    s36 = jnp.concatenate([sd, sr], axis=1)
    ld = s_all[:, :, 6:9].transpose(1, 0, 2).reshape(P, 18)
    lr = s_all[:, :, 9:12].transpose(1, 0, 2).reshape(P, 18)
    sself = jnp.concatenate([ld, lr], axis=1)

    nb2 = P // BP2
    pm18, rgb18 = pl.pallas_call(
        _stage2_body,
        grid=(nb2,),
        in_specs=[
            pl.BlockSpec((BP2, P), lambda j: (j, 0)),
            pl.BlockSpec((P, 36), lambda j: (0, 0)),
            pl.BlockSpec((BP2, 36), lambda j: (j, 0)),
        ],
        out_specs=[
            pl.BlockSpec((BP2, 18), lambda j: (j, 0)),
            pl.BlockSpec((BP2, 18), lambda j: (j, 0)),
        ],
        out_shape=[
            jax.ShapeDtypeStruct((P, 18), f32),
            jax.ShapeDtypeStruct((P, 18), f32),
        ],
    )(adj.astype(f32), s36, sself)

    points_move = pm18.reshape(P, B, 3).transpose(1, 0, 2)
    rgb = rgb18.reshape(P, B, 3).transpose(1, 0, 2)
    return (points_move, rgb)
